# baseline (speedup 1.0000x reference)
import functools

import jax
import jax.numpy as jnp
from jax import lax
from jax.experimental import pallas as pl
from jax.experimental.pallas import tpu as pltpu
from jax.experimental.pallas import tpu_sc as plsc

_N = 10000
_E = 320000
_CH = 64
_K = 20
_NB = 5
_NA = _N + 1
_EP = 327680
_NCH = _EP // _CH
_NC = 2
_NS = 16
_W = 128
_WB = 632
_WLAST = _N - (_NS - 1) * _WB


def _fill1d(ref, n, val):
    def body(i, carry):
        ref[pl.ds(pl.multiple_of(i * 16, 16), 16)] = jnp.full(
            (16,), val, jnp.float32)
        return carry

    lax.fori_loop(0, n // 16, body, 0, unroll=False)


def _rows8(s):
    return pl.multiple_of(s * _WB, 8)


def _init_and_out(zz_hbm, acc, out_hbm, s, out_base, do_init):
    r0 = _rows8(s)

    @pl.when(s < _NS - 1)
    def _():
        if do_init:
            pltpu.sync_copy(zz_hbm, acc.at[pl.ds(r0, _WB)])
        else:
            pltpu.sync_copy(acc.at[pl.ds(r0, _WB)],
                            out_hbm.at[pl.ds(out_base + r0, _WB)])

    @pl.when(s == _NS - 1)
    def _():
        if do_init:
            pltpu.sync_copy(zz_hbm.at[pl.ds(0, _WLAST)],
                            acc.at[pl.ds(r0, _WLAST)])
        else:
            pltpu.sync_copy(acc.at[pl.ds(r0, _WLAST)],
                            out_hbm.at[pl.ds(out_base + r0, _WLAST)])


def _pipeline(gather_ref, idx_src, idx_dst, rows_v, acc, semg, sems,
              deg_fn=None):
    gat = [None] * _K
    sca = [None] * _K
    for b in range(_NB):
        gat[b] = pltpu.async_copy(
            gather_ref.at[idx_src.at[b]], rows_v.at[b % _NB], semg[b % _NB])
    for t in range(_K):
        gat[t].wait()
        sca[t] = pltpu.async_copy(
            rows_v.at[t % _NB], acc.at[idx_dst.at[t]], sems[t % _NB],
            add=True)
        if deg_fn is not None:
            deg_fn(t)
        d = max(1, _NB - 2)
        u = t - d + _NB
        if t >= d and u < _K:
            sca[t - d].wait()
            gat[u] = pltpu.async_copy(
                gather_ref.at[idx_src.at[u]],
                rows_v.at[u % _NB], semg[u % _NB])
    for t in range(_K - _NB, _K):
        sca[t].wait()


def _segsum_layer1(x, src3, dst3, zz):
    mesh = plsc.VectorSubcoreMesh(
        core_axis_name="c", subcore_axis_name="s", num_cores=_NC,
        num_subcores=_NS)
    cht = _NCH // (_NC * _NS)

    @functools.partial(
        pl.kernel,
        out_type=(
            jax.ShapeDtypeStruct((2 * _N, _W), jnp.float32),
            jax.ShapeDtypeStruct((2 * _N,), jnp.float32),
        ),
        mesh=mesh,
        scratch_types=[
            pltpu.VMEM((_K, _CH), jnp.int32),
            pltpu.VMEM((_K, _CH), jnp.int32),
            pltpu.VMEM((_NB, _CH, _W), jnp.float32),
            pltpu.VMEM((_CH,), jnp.float32),
            pltpu.VMEM((640,), jnp.float32),
            pltpu.VMEM_SHARED((_NA, _W), jnp.float32),
            pltpu.VMEM_SHARED((_NA,), jnp.float32),
            [pltpu.SemaphoreType.DMA] * _NB,
            [pltpu.SemaphoreType.DMA] * _NB,
            pltpu.SemaphoreType.DMA,
        ],
    )
    def k(x_hbm, src_hbm, dst_hbm, zz_hbm, p_hbm, dd_hbm,
          src_v, dst_v, rows_v, ones_v, zd_v, acc, accd,
          semg, sems, semd):
        c = lax.axis_index("c")
        s = lax.axis_index("s")
        w = c * _NS + s
        _fill1d(ones_v, _CH, 1.0)
        _fill1d(zd_v, 640, 0.0)
        _init_and_out(zz_hbm, acc, None, s, 0, True)
        r0 = _rows8(s)

        @pl.when(s < _NS - 1)
        def _():
            pltpu.sync_copy(zd_v.at[pl.ds(0, _WB)], accd.at[pl.ds(r0, _WB)])

        @pl.when(s == _NS - 1)
        def _():
            pltpu.sync_copy(zd_v.at[pl.ds(0, _WLAST)],
                            accd.at[pl.ds(r0, _WLAST)])

        plsc.subcore_barrier()

        def group(g, carry):
            pltpu.sync_copy(src_hbm.at[w, g], src_v)
            pltpu.sync_copy(dst_hbm.at[w, g], dst_v)
            deg = [None] * _K

            def deg_fn(t):
                deg[t] = pltpu.async_copy(
                    ones_v, accd.at[dst_v.at[t]], semd, add=True)
                if t >= 8:
                    deg[t - 8].wait()

            _pipeline(x_hbm, src_v, dst_v, rows_v, acc, semg, sems, deg_fn)
            for t in range(_K - 8, _K):
                deg[t].wait()
            return carry

        lax.fori_loop(0, cht // _K, group, 0, unroll=False)
        plsc.subcore_barrier()
        _init_and_out(None, acc, p_hbm, s, c * _N, False)

        @pl.when(s < _NS - 1)
        def _():
            pltpu.sync_copy(accd.at[pl.ds(r0, _WB)], zd_v.at[pl.ds(0, _WB)])
            pltpu.sync_copy(zd_v.at[pl.ds(0, _WB)],
                            dd_hbm.at[pl.ds(c * _N + r0, _WB)])

        @pl.when(s == _NS - 1)
        def _():
            pltpu.sync_copy(accd.at[pl.ds(r0, _WLAST)],
                            zd_v.at[pl.ds(0, _WLAST)])
            pltpu.sync_copy(zd_v.at[pl.ds(0, _WLAST)],
                            dd_hbm.at[pl.ds(c * _N + r0, _WLAST)])

    return k(x, src3, dst3, zz)


def _segsum_half(tabs, src3, dst3, zz):
    mesh = plsc.VectorSubcoreMesh(
        core_axis_name="c", subcore_axis_name="s", num_cores=_NC,
        num_subcores=_NS)
    cht = _NCH // _NS

    @functools.partial(
        pl.kernel,
        out_type=jax.ShapeDtypeStruct((2 * _N, _W), jnp.float32),
        mesh=mesh,
        scratch_types=[
            pltpu.VMEM((_K, _CH), jnp.int32),
            pltpu.VMEM((_K, _CH), jnp.int32),
            pltpu.VMEM((_NB, _CH, _W), jnp.float32),
            pltpu.VMEM_SHARED((_NA, _W), jnp.float32),
            [pltpu.SemaphoreType.DMA] * _NB,
            [pltpu.SemaphoreType.DMA] * _NB,
        ],
    )
    def k(tabs_hbm, src_hbm, dst_hbm, zz_hbm, q_hbm,
          src_v, dst_v, rows_v, acc, semg, sems):
        c = lax.axis_index("c")
        s = lax.axis_index("s")
        _init_and_out(zz_hbm, acc, None, s, 0, True)
        plsc.subcore_barrier()

        def group(g, carry):
            pltpu.sync_copy(src_hbm.at[c, s, g], src_v)
            pltpu.sync_copy(dst_hbm.at[s, g], dst_v)
            _pipeline(tabs_hbm, src_v, dst_v, rows_v, acc, semg, sems)
            return carry

        lax.fori_loop(0, cht // _K, group, 0, unroll=False)
        plsc.subcore_barrier()
        _init_and_out(None, acc, q_hbm, s, c * _N, False)

    return k(tabs, src3, dst3, zz)


_R = 1000
_G = _N // _R


def _dot(a, b):
    return jax.lax.dot_general(a, b, (((1,), (0,)), ((), ())),
                               preferred_element_type=jnp.float32)


def _tc_layer1(x, p, dd, ws, wn, b):
    def body(x_ref, pa_ref, pb_ref, da_ref, db_ref, ws_ref, wn_ref, b_ref,
             h_ref, m_ref):
        i = pl.program_id(0)
        d = jnp.maximum(da_ref[...] + db_ref[...], 1.0)
        rcp = 1.0 / d
        agg = (pa_ref[...] + pb_ref[...]) * rcp
        h = _dot(x_ref[...], ws_ref[...]) + _dot(agg, wn_ref[...]) + b_ref[...]
        h = jnp.maximum(h, 0.0)
        h_ref[0] = h[:, :_W]
        h_ref[1] = h[:, _W:]
        cur = jnp.max(h, axis=0, keepdims=True)

        @pl.when(i == 0)
        def _():
            m_ref[...] = cur

        @pl.when(i > 0)
        def _():
            m_ref[...] = jnp.maximum(m_ref[...], cur)

    return pl.pallas_call(
        body,
        grid=(_G,),
        in_specs=[
            pl.BlockSpec((_R, _W), lambda i: (i, 0)),
            pl.BlockSpec((_R, _W), lambda i: (i, 0)),
            pl.BlockSpec((_R, _W), lambda i: (_G + i, 0)),
            pl.BlockSpec((_R, 1), lambda i: (i, 0)),
            pl.BlockSpec((_R, 1), lambda i: (_G + i, 0)),
            pl.BlockSpec((_W, 2 * _W), lambda i: (0, 0)),
            pl.BlockSpec((_W, 2 * _W), lambda i: (0, 0)),
            pl.BlockSpec((1, 2 * _W), lambda i: (0, 0)),
        ],
        out_specs=[
            pl.BlockSpec((2, _R, _W), lambda i: (0, i, 0)),
            pl.BlockSpec((1, 2 * _W), lambda i: (0, 0)),
        ],
        out_shape=[
            jax.ShapeDtypeStruct((2, _N, _W), jnp.float32),
            jax.ShapeDtypeStruct((1, 2 * _W), jnp.float32),
        ],
    )(x, p, p, dd, dd, ws, wn, b)


def _tc_layer23(t, q, dd, ws, wn, b):
    def body(ta_ref, tb_ref, qa_ref, qb_ref, da_ref, db_ref,
             wsa_ref, wsb_ref, wna_ref, wnb_ref, b_ref, h_ref, m_ref):
        i = pl.program_id(0)
        d = jnp.maximum(da_ref[...] + db_ref[...], 1.0)
        rcp = 1.0 / d
        h = (_dot(ta_ref[...], wsa_ref[...]) + _dot(tb_ref[...], wsb_ref[...])
             + _dot(qa_ref[...] * rcp, wna_ref[...])
             + _dot(qb_ref[...] * rcp, wnb_ref[...]) + b_ref[...])
        h = jnp.maximum(h, 0.0)
        h_ref[0] = h[:, :_W]
        h_ref[1] = h[:, _W:]
        cur = jnp.max(h, axis=0, keepdims=True)

        @pl.when(i == 0)
        def _():
            m_ref[...] = cur

        @pl.when(i > 0)
        def _():
            m_ref[...] = jnp.maximum(m_ref[...], cur)

    return pl.pallas_call(
        body,
        grid=(_G,),
        in_specs=[
            pl.BlockSpec((_R, _W), lambda i: (i, 0)),
            pl.BlockSpec((_R, _W), lambda i: (_G + i, 0)),
            pl.BlockSpec((_R, _W), lambda i: (i, 0)),
            pl.BlockSpec((_R, _W), lambda i: (_G + i, 0)),
            pl.BlockSpec((_R, 1), lambda i: (i, 0)),
            pl.BlockSpec((_R, 1), lambda i: (_G + i, 0)),
            pl.BlockSpec((_W, 2 * _W), lambda i: (0, 0)),
            pl.BlockSpec((_W, 2 * _W), lambda i: (0, 0)),
            pl.BlockSpec((_W, 2 * _W), lambda i: (0, 0)),
            pl.BlockSpec((_W, 2 * _W), lambda i: (0, 0)),
            pl.BlockSpec((1, 2 * _W), lambda i: (0, 0)),
        ],
        out_specs=[
            pl.BlockSpec((2, _R, _W), lambda i: (0, i, 0)),
            pl.BlockSpec((1, 2 * _W), lambda i: (0, 0)),
        ],
        out_shape=[
            jax.ShapeDtypeStruct((2, _N, _W), jnp.float32),
            jax.ShapeDtypeStruct((1, 2 * _W), jnp.float32),
        ],
    )(t, t, q, q, dd, dd, ws[:_W], ws[_W:], wn[:_W], wn[_W:], b)


def kernel(node_feat, edge_index, W_self0, W_neigh0, b0,
           W_self1, W_neigh1, b1, W_self2, W_neigh2, b2):
    npad = _EP - _E
    src = jnp.concatenate([edge_index[0], jnp.zeros((npad,), jnp.int32)])
    dst = jnp.concatenate([edge_index[1], jnp.full((npad,), _N, jnp.int32)])
    src3a = src.reshape(_NC * _NS, -1, _K, _CH)
    dst3a = dst.reshape(_NC * _NS, -1, _K, _CH)
    src3b = jnp.stack([src.reshape(_NS, -1, _K, _CH),
                       (src + _N).reshape(_NS, -1, _K, _CH)])
    dst3b = dst.reshape(_NS, -1, _K, _CH)
    zz = jnp.zeros((_WB, _W), jnp.float32)

    p, dd1 = _segsum_layer1(node_feat, src3a, dst3a, zz)
    dd = dd1.reshape(2 * _N, 1)
    h1, m1 = _tc_layer1(node_feat, p, dd, W_self0, W_neigh0,
                        b0.reshape(1, -1))
    t1 = h1.reshape(2 * _N, _W)

    q2 = _segsum_half(t1, src3b, dst3b, zz)
    h2, m2 = _tc_layer23(t1, q2, dd, W_self1, W_neigh1, b1.reshape(1, -1))
    t2 = h2.reshape(2 * _N, _W)

    q3 = _segsum_half(t2, src3b, dst3b, zz)
    _, m3 = _tc_layer23(t2, q3, dd, W_self2, W_neigh2, b2.reshape(1, -1))

    return jnp.concatenate([m1[0], m2[0], m3[0]])


# speedup vs baseline: 3.2399x; 3.2399x over previous
import functools

import jax
import jax.numpy as jnp
from jax import lax
from jax.experimental import pallas as pl
from jax.experimental.pallas import tpu as pltpu
from jax.experimental.pallas import tpu_sc as plsc

_N = 10000
_E = 320000
_CH = 80
_K = 25
_NB = 3
_NCH = _E // _CH
_NC = 2
_NS = 16
_W = 128
_WB = 632
_WLAST = _N - (_NS - 1) * _WB


def _fill1d(ref, n, val):
    def body(i, carry):
        ref[pl.ds(pl.multiple_of(i * 16, 16), 16)] = jnp.full(
            (16,), val, jnp.float32)
        return carry

    lax.fori_loop(0, n // 16, body, 0, unroll=False)


def _rows8(s):
    return pl.multiple_of(s * _WB, 8)


def _init_and_out(zz_hbm, acc, out_hbm, s, out_base, do_init):
    r0 = _rows8(s)

    @pl.when(s < _NS - 1)
    def _():
        if do_init:
            pltpu.sync_copy(zz_hbm, acc.at[pl.ds(r0, _WB)])
        else:
            pltpu.sync_copy(acc.at[pl.ds(r0, _WB)],
                            out_hbm.at[pl.ds(out_base + r0, _WB)])

    @pl.when(s == _NS - 1)
    def _():
        if do_init:
            pltpu.sync_copy(zz_hbm.at[pl.ds(0, _WLAST)],
                            acc.at[pl.ds(r0, _WLAST)])
        else:
            pltpu.sync_copy(acc.at[pl.ds(r0, _WLAST)],
                            out_hbm.at[pl.ds(out_base + r0, _WLAST)])


def _run_all(nch, src_load, dst_load, gather_tab, src_v, dst_v, rows_v, acc,
             semg, sems, deg_fn=None, deg_end=None):
    ng = nch // _K
    d = max(1, _NB - 2)
    idx_d = [None] * ng
    idx_d[0] = src_load(0, 0) + dst_load(0, 0)
    idx_waited = [False] * ng

    def ensure_idx(g):
        if not idx_waited[g]:
            for dd in idx_d[g]:
                dd.wait()
            idx_waited[g] = True

    def sv(j):
        return src_v.at[(j // _K) % 2].at[j % _K]

    def dv(j):
        return dst_v.at[(j // _K) % 2].at[j % _K]

    gat = [None] * nch
    sca = [None] * nch

    def issue_gather(u):
        ensure_idx(u // _K)
        gat[u] = pltpu.async_copy(
            gather_tab.at[sv(u)], rows_v.at[u % _NB], semg[u % _NB])

    for b in range(_NB):
        issue_gather(b)
    for t in range(nch):
        gat[t].wait()
        sca[t] = pltpu.async_copy(
            rows_v.at[t % _NB], acc.at[dv(t)], sems[t % _NB], add=True)
        if deg_fn is not None:
            deg_fn(t, dv(t))
        if t % _K == 1 and t // _K + 1 < ng:
            g1 = t // _K + 1
            idx_d[g1] = src_load(g1, g1 % 2) + dst_load(g1, g1 % 2)
        u = t - d + _NB
        if t >= d and u < nch:
            sca[t - d].wait()
            issue_gather(u)
    for t in range(nch - _NB, nch):
        sca[t].wait()
    if deg_end is not None:
        deg_end()


def _segsum_layer1(x, src3, dst3, zz):
    mesh = plsc.VectorSubcoreMesh(
        core_axis_name="c", subcore_axis_name="s", num_cores=_NC,
        num_subcores=_NS)
    cht = _NCH // (_NC * _NS)

    @functools.partial(
        pl.kernel,
        out_type=(
            jax.ShapeDtypeStruct((2 * _N, _W), jnp.float32),
            jax.ShapeDtypeStruct((2 * _N,), jnp.float32),
        ),
        mesh=mesh,
        scratch_types=[
            pltpu.VMEM((2, _K, _CH), jnp.int32),
            pltpu.VMEM((2, _K, _CH), jnp.int32),
            pltpu.VMEM((_NB, _CH, _W), jnp.float32),
            pltpu.VMEM((_CH,), jnp.float32),
            pltpu.VMEM((640,), jnp.float32),
            pltpu.VMEM_SHARED((_N, _W), jnp.float32),
            pltpu.VMEM_SHARED((_N,), jnp.float32),
            [pltpu.SemaphoreType.DMA] * _NB,
            [pltpu.SemaphoreType.DMA] * _NB,
            pltpu.SemaphoreType.DMA,
            pltpu.SemaphoreType.DMA,
        ],
    )
    def k(x_hbm, src_hbm, dst_hbm, zz_hbm, p_hbm, dd_hbm,
          src_v, dst_v, rows_v, ones_v, zd_v, acc, accd,
          semg, sems, semd, semi):
        c = lax.axis_index("c")
        s = lax.axis_index("s")
        w = c * _NS + s
        _fill1d(ones_v, _CH, 1.0)
        _fill1d(zd_v, 640, 0.0)
        _init_and_out(zz_hbm, acc, None, s, 0, True)
        r0 = _rows8(s)

        @pl.when(s < _NS - 1)
        def _():
            pltpu.sync_copy(zd_v.at[pl.ds(0, _WB)], accd.at[pl.ds(r0, _WB)])

        @pl.when(s == _NS - 1)
        def _():
            pltpu.sync_copy(zd_v.at[pl.ds(0, _WLAST)],
                            accd.at[pl.ds(r0, _WLAST)])

        plsc.subcore_barrier()

        def src_load(g, buf):
            return [pltpu.async_copy(src_hbm.at[w, g], src_v.at[buf], semi)]

        def dst_load(g, buf):
            return [pltpu.async_copy(dst_hbm.at[w, g], dst_v.at[buf], semi)]

        deg = [None] * cht

        def deg_fn(t, dvt):
            deg[t] = pltpu.async_copy(ones_v, accd.at[dvt], semd, add=True)
            if t >= 8:
                deg[t - 8].wait()

        def deg_end():
            for t in range(cht - 8, cht):
                deg[t].wait()

        _run_all(cht, src_load, dst_load, x_hbm, src_v, dst_v, rows_v, acc,
                 semg, sems, deg_fn, deg_end)
        plsc.subcore_barrier()
        _init_and_out(None, acc, p_hbm, s, c * _N, False)

        @pl.when(s < _NS - 1)
        def _():
            pltpu.sync_copy(accd.at[pl.ds(r0, _WB)], zd_v.at[pl.ds(0, _WB)])
            pltpu.sync_copy(zd_v.at[pl.ds(0, _WB)],
                            dd_hbm.at[pl.ds(c * _N + r0, _WB)])

        @pl.when(s == _NS - 1)
        def _():
            pltpu.sync_copy(accd.at[pl.ds(r0, _WLAST)],
                            zd_v.at[pl.ds(0, _WLAST)])
            pltpu.sync_copy(zd_v.at[pl.ds(0, _WLAST)],
                            dd_hbm.at[pl.ds(c * _N + r0, _WLAST)])

    return k(x, src3, dst3, zz)


def _segsum_half(tabs, src3, dst3, zz):
    mesh = plsc.VectorSubcoreMesh(
        core_axis_name="c", subcore_axis_name="s", num_cores=_NC,
        num_subcores=_NS)
    cht = _NCH // _NS

    @functools.partial(
        pl.kernel,
        out_type=jax.ShapeDtypeStruct((2 * _N, _W), jnp.float32),
        mesh=mesh,
        scratch_types=[
            pltpu.VMEM((2, _K, _CH), jnp.int32),
            pltpu.VMEM((2, _K, _CH), jnp.int32),
            pltpu.VMEM((_NB, _CH, _W), jnp.float32),
            pltpu.VMEM_SHARED((_N, _W), jnp.float32),
            [pltpu.SemaphoreType.DMA] * _NB,
            [pltpu.SemaphoreType.DMA] * _NB,
            pltpu.SemaphoreType.DMA,
        ],
    )
    def k(tabs_hbm, src_hbm, dst_hbm, zz_hbm, q_hbm,
          src_v, dst_v, rows_v, acc, semg, sems, semi):
        c = lax.axis_index("c")
        s = lax.axis_index("s")
        _init_and_out(zz_hbm, acc, None, s, 0, True)
        plsc.subcore_barrier()

        def src_load(g, buf):
            return [pltpu.async_copy(src_hbm.at[c, s, g], src_v.at[buf],
                                     semi)]

        def dst_load(g, buf):
            return [pltpu.async_copy(dst_hbm.at[s, g], dst_v.at[buf], semi)]

        _run_all(cht, src_load, dst_load, tabs_hbm, src_v, dst_v, rows_v,
                 acc, semg, sems)
        plsc.subcore_barrier()
        _init_and_out(None, acc, q_hbm, s, c * _N, False)

    return k(tabs, src3, dst3, zz)


_R = 1000
_G = _N // _R


def _dot(a, b):
    return jax.lax.dot_general(a, b, (((1,), (0,)), ((), ())),
                               preferred_element_type=jnp.float32)


def _tc_layer1(x, p, dd, ws, wn, b):
    def body(x_ref, pa_ref, pb_ref, da_ref, db_ref, ws_ref, wn_ref, b_ref,
             h_ref, m_ref):
        i = pl.program_id(0)
        d = jnp.maximum(da_ref[...] + db_ref[...], 1.0)
        rcp = 1.0 / d
        agg = (pa_ref[...] + pb_ref[...]) * rcp
        h = _dot(x_ref[...], ws_ref[...]) + _dot(agg, wn_ref[...]) + b_ref[...]
        h = jnp.maximum(h, 0.0)
        h_ref[0] = h[:, :_W]
        h_ref[1] = h[:, _W:]
        cur = jnp.max(h, axis=0, keepdims=True)

        @pl.when(i == 0)
        def _():
            m_ref[...] = cur

        @pl.when(i > 0)
        def _():
            m_ref[...] = jnp.maximum(m_ref[...], cur)

    return pl.pallas_call(
        body,
        grid=(_G,),
        in_specs=[
            pl.BlockSpec((_R, _W), lambda i: (i, 0)),
            pl.BlockSpec((_R, _W), lambda i: (i, 0)),
            pl.BlockSpec((_R, _W), lambda i: (_G + i, 0)),
            pl.BlockSpec((_R, 1), lambda i: (i, 0)),
            pl.BlockSpec((_R, 1), lambda i: (_G + i, 0)),
            pl.BlockSpec((_W, 2 * _W), lambda i: (0, 0)),
            pl.BlockSpec((_W, 2 * _W), lambda i: (0, 0)),
            pl.BlockSpec((1, 2 * _W), lambda i: (0, 0)),
        ],
        out_specs=[
            pl.BlockSpec((2, _R, _W), lambda i: (0, i, 0)),
            pl.BlockSpec((1, 2 * _W), lambda i: (0, 0)),
        ],
        out_shape=[
            jax.ShapeDtypeStruct((2, _N, _W), jnp.float32),
            jax.ShapeDtypeStruct((1, 2 * _W), jnp.float32),
        ],
    )(x, p, p, dd, dd, ws, wn, b)


def _tc_layer23(t, q, dd, ws, wn, b):
    def body(ta_ref, tb_ref, qa_ref, qb_ref, da_ref, db_ref,
             wsa_ref, wsb_ref, wna_ref, wnb_ref, b_ref, h_ref, m_ref):
        i = pl.program_id(0)
        d = jnp.maximum(da_ref[...] + db_ref[...], 1.0)
        rcp = 1.0 / d
        h = (_dot(ta_ref[...], wsa_ref[...]) + _dot(tb_ref[...], wsb_ref[...])
             + _dot(qa_ref[...] * rcp, wna_ref[...])
             + _dot(qb_ref[...] * rcp, wnb_ref[...]) + b_ref[...])
        h = jnp.maximum(h, 0.0)
        h_ref[0] = h[:, :_W]
        h_ref[1] = h[:, _W:]
        cur = jnp.max(h, axis=0, keepdims=True)

        @pl.when(i == 0)
        def _():
            m_ref[...] = cur

        @pl.when(i > 0)
        def _():
            m_ref[...] = jnp.maximum(m_ref[...], cur)

    return pl.pallas_call(
        body,
        grid=(_G,),
        in_specs=[
            pl.BlockSpec((_R, _W), lambda i: (i, 0)),
            pl.BlockSpec((_R, _W), lambda i: (_G + i, 0)),
            pl.BlockSpec((_R, _W), lambda i: (i, 0)),
            pl.BlockSpec((_R, _W), lambda i: (_G + i, 0)),
            pl.BlockSpec((_R, 1), lambda i: (i, 0)),
            pl.BlockSpec((_R, 1), lambda i: (_G + i, 0)),
            pl.BlockSpec((_W, 2 * _W), lambda i: (0, 0)),
            pl.BlockSpec((_W, 2 * _W), lambda i: (0, 0)),
            pl.BlockSpec((_W, 2 * _W), lambda i: (0, 0)),
            pl.BlockSpec((_W, 2 * _W), lambda i: (0, 0)),
            pl.BlockSpec((1, 2 * _W), lambda i: (0, 0)),
        ],
        out_specs=[
            pl.BlockSpec((2, _R, _W), lambda i: (0, i, 0)),
            pl.BlockSpec((1, 2 * _W), lambda i: (0, 0)),
        ],
        out_shape=[
            jax.ShapeDtypeStruct((2, _N, _W), jnp.float32),
            jax.ShapeDtypeStruct((1, 2 * _W), jnp.float32),
        ],
    )(t, t, q, q, dd, dd, ws[:_W], ws[_W:], wn[:_W], wn[_W:], b)


def kernel(node_feat, edge_index, W_self0, W_neigh0, b0,
           W_self1, W_neigh1, b1, W_self2, W_neigh2, b2):
    src = edge_index[0]
    dst = edge_index[1]
    src3a = src.reshape(_NC * _NS, -1, _K, _CH)
    dst3a = dst.reshape(_NC * _NS, -1, _K, _CH)
    src3b = jnp.stack([src.reshape(_NS, -1, _K, _CH),
                       (src + _N).reshape(_NS, -1, _K, _CH)])
    dst3b = dst.reshape(_NS, -1, _K, _CH)
    zz = jnp.zeros((_WB, _W), jnp.float32)

    p, dd1 = _segsum_layer1(node_feat, src3a, dst3a, zz)
    dd = dd1.reshape(2 * _N, 1)
    h1, m1 = _tc_layer1(node_feat, p, dd, W_self0, W_neigh0,
                        b0.reshape(1, -1))
    t1 = h1.reshape(2 * _N, _W)

    q2 = _segsum_half(t1, src3b, dst3b, zz)
    h2, m2 = _tc_layer23(t1, q2, dd, W_self1, W_neigh1, b1.reshape(1, -1))
    t2 = h2.reshape(2 * _N, _W)

    q3 = _segsum_half(t2, src3b, dst3b, zz)
    _, m3 = _tc_layer23(t2, q3, dd, W_self2, W_neigh2, b2.reshape(1, -1))

    return jnp.concatenate([m1[0], m2[0], m3[0]])
